# baseline (speedup 1.0000x reference)
import jax
import jax.numpy as jnp
from jax import lax
from jax.experimental import pallas as pl
from jax.experimental.pallas import tpu as pltpu
from jax.experimental.pallas import tpu_sc as plsc

BATCH = 1024
BW = BATCH // 2
TAU = 30.0
NCLS = 10
NC, NS, L = 2, 16, 16
NW = NC * NS
OUT_PAD = 8192
BPW = OUT_PAD // NW
GRP = 16
NGRP = BPW // GRP
RQ = 4

_GATE_K = (
    (0, 0, 0, 0), (0, 0, 0, 1), (0, 1, 0, -1), (0, 1, 0, 0),
    (0, 0, 1, -1), (0, 0, 1, 0), (0, 1, 1, -2), (0, 1, 1, -1),
    (1, -1, -1, 1), (1, -1, -1, 2), (1, 0, -1, 0), (1, 0, -1, 1),
    (1, -1, 0, 0), (1, -1, 0, 1), (1, 0, 0, -1), (1, 0, 0, 0),
)


def _coef_tc(wall):
    def body(w_ref, k_ref, o_ref):
        w = w_ref[...]
        m = jnp.max(w, axis=-1, keepdims=True)
        e = jnp.exp(w - m)
        p = e / jnp.sum(e, axis=-1, keepdims=True)
        o_ref[...] = jax.lax.dot(p, k_ref[...], precision=lax.Precision.HIGHEST)

    n = wall.shape[0]
    blk = 2048
    return pl.pallas_call(
        body,
        grid=(n // blk,),
        in_specs=[
            pl.BlockSpec((blk, 16), lambda i: (i, 0)),
            pl.BlockSpec((16, 4), lambda i: (0, 0)),
        ],
        out_specs=pl.BlockSpec((blk, 4), lambda i: (i, 0)),
        out_shape=jax.ShapeDtypeStruct((n, 4), jnp.float32),
    )(wall, jnp.asarray(_GATE_K, dtype=jnp.float32))


def _sc_layer(table, aidx, bidx, cfs):
    mesh = plsc.VectorSubcoreMesh(core_axis_name="c", subcore_axis_name="s")

    def body(tab, ai, bi, cf, out, aiv, biv,
             abufs, bbufs, obufs, cbufs, sems_a, sems_b, sems_o, sems_c):
        wid = lax.axis_index("s") * NC + lax.axis_index("c")
        base = wid * BPW
        pltpu.sync_copy(ai.at[pl.ds(base, BPW)], aiv)
        pltpu.sync_copy(bi.at[pl.ds(base, BPW)], biv)
        W = 2 * L

        def issue(g, s):
            r0 = pl.multiple_of(g * GRP, GRP)
            pltpu.async_copy(
                tab.at[aiv.at[pl.ds(r0, GRP)]], abufs[s], sems_a[s])
            pltpu.async_copy(
                tab.at[biv.at[pl.ds(r0, GRP)]], bbufs[s], sems_b[s])
            pltpu.async_copy(
                cf.at[pl.ds(base + r0, GRP)], cbufs[s], sems_c[s])

        def wait_in(s):
            pltpu.make_async_copy(
                tab.at[aiv.at[pl.ds(0, GRP)]], abufs[s], sems_a[s]).wait()
            pltpu.make_async_copy(
                tab.at[biv.at[pl.ds(0, GRP)]], bbufs[s], sems_b[s]).wait()
            pltpu.make_async_copy(
                cf.at[pl.ds(base, GRP)], cbufs[s], sems_c[s]).wait()

        def wait_out(s):
            pltpu.make_async_copy(
                obufs[s], out.at[pl.ds(base, GRP)], sems_o[s]).wait()

        def do_group(g, s):
            abufb = abufs[s].bitcast(jnp.bfloat16)
            bbufb = bbufs[s].bitcast(jnp.bfloat16)
            obufb = obufs[s].bitcast(jnp.bfloat16)
            cbufb = cbufs[s].bitcast(jnp.bfloat16)
            r0 = pl.multiple_of(g * GRP, GRP)
            for q in range(GRP // RQ):
                rows = [q * RQ + i for i in range(RQ)]
                cs = [(cbufb[2 * r, pl.ds(0, W)],
                       cbufb[2 * r, pl.ds(W, W)],
                       cbufb[2 * r + 1, pl.ds(0, W)],
                       cbufb[2 * r + 1, pl.ds(W, W)]) for r in rows]

                def col_fn(j, carry2, rows=rows, cs=cs,
                           abufb=abufb, bbufb=bbufb, obufb=obufb):
                    sl = pl.ds(j * W, W)
                    for r, (c0, c1, c2, c3) in zip(rows, cs):
                        for v in (2 * r, 2 * r + 1):
                            av = abufb[v, sl]
                            bv = bbufb[v, sl]
                            obufb[v, sl] = (c0 + c1 * av) + (c2 + c3 * av) * bv
                    return carry2

                lax.fori_loop(0, BW // W, col_fn, 0)
            pltpu.async_copy(
                obufs[s], out.at[pl.ds(base + r0, GRP)], sems_o[s])

        issue(0, 0)

        def pair_fn(h, carry):
            for s in (0, 1):
                g = 2 * h + s

                @pl.when(g + 1 < NGRP)
                def _():
                    issue(g + 1, 1 - s)

                wait_in(s)

                @pl.when(g >= 2)
                def _():
                    wait_out(s)

                do_group(g, s)
            return carry

        lax.fori_loop(0, NGRP // 2, pair_fn, 0)
        wait_out(0)
        wait_out(1)

    kfn = pl.kernel(
        body,
        out_type=jax.ShapeDtypeStruct((OUT_PAD, BW), jnp.int32),
        mesh=mesh,
        scratch_types=[
            pltpu.VMEM((BPW,), jnp.int32),
            pltpu.VMEM((BPW,), jnp.int32),
            [pltpu.VMEM((GRP, BW), jnp.int32)] * 2,
            [pltpu.VMEM((GRP, BW), jnp.int32)] * 2,
            [pltpu.VMEM((GRP, BW), jnp.int32)] * 2,
            [pltpu.VMEM((GRP, 4 * L), jnp.int32)] * 2,
            [pltpu.SemaphoreType.DMA] * 2,
            [pltpu.SemaphoreType.DMA] * 2,
            [pltpu.SemaphoreType.DMA] * 2,
            [pltpu.SemaphoreType.DMA] * 2,
        ],
    )
    return kfn(table, aidx, bidx, cfs)


def _gsum_tc(y, n_valid):
    rows = n_valid // NCLS

    def body(y_ref, o_ref):
        yw = y_ref[...]
        lo = jax.lax.bitcast_convert_type(yw << 16, jnp.float32)
        hi = jax.lax.bitcast_convert_type(
            yw & jnp.int32(-65536), jnp.float32)
        o_ref[0, 0, :] = jnp.sum(lo, axis=0) / TAU
        o_ref[0, 1, :] = jnp.sum(hi, axis=0) / TAU

    return pl.pallas_call(
        body,
        grid=(NCLS,),
        in_specs=[pl.BlockSpec((rows, BW), lambda c: (c, 0))],
        out_specs=pl.BlockSpec((1, 2, BW), lambda c: (c, 0, 0)),
        out_shape=jax.ShapeDtypeStruct((NCLS, 2, BW), jnp.float32),
    )(y)


def kernel(x, w1, w2, w3, a1, b1, a2, b2, a3, b3):
    xtb = x.T.astype(jnp.bfloat16)
    xt = jax.lax.bitcast_convert_type(
        xtb.reshape(x.shape[1], BW, 2), jnp.int32)
    n3 = w3.shape[0]
    w3p = jnp.concatenate([w3, jnp.zeros((OUT_PAD - n3, 16), jnp.float32)], 0)
    wall = jnp.concatenate([w1, w2, w3p], axis=0)
    coefs = _coef_tc(wall)
    n = coefs.shape[0]
    cb = coefs.astype(jnp.bfloat16)
    W = 2 * L
    lo = jnp.concatenate(
        [jnp.broadcast_to(cb[:, 0:1], (n, W)),
         jnp.broadcast_to(cb[:, 1:2], (n, W))], axis=1)
    hi = jnp.concatenate(
        [jnp.broadcast_to(cb[:, 2:3], (n, W)),
         jnp.broadcast_to(cb[:, 3:4], (n, W))], axis=1)
    cfp = jax.lax.bitcast_convert_type(
        jnp.stack([lo, hi], axis=-1), jnp.int32)
    cf1 = cfp[:OUT_PAD]
    cf2 = cfp[OUT_PAD:2 * OUT_PAD]
    cf3 = cfp[2 * OUT_PAD:]
    padi = jnp.zeros((OUT_PAD - n3,), jnp.int32)
    a3p = jnp.concatenate([a3, padi])
    b3p = jnp.concatenate([b3, padi])
    y1 = _sc_layer(xt, a1, b1, cf1)
    y2 = _sc_layer(y1, a2, b2, cf2)
    y3 = _sc_layer(y2, a3p, b3p, cf3)
    cls = _gsum_tc(y3, n3)
    return cls.transpose(2, 1, 0).reshape(BATCH, NCLS)


# speedup vs baseline: 1.3613x; 1.3613x over previous
import jax
import jax.numpy as jnp
from jax import lax
from jax.experimental import pallas as pl
from jax.experimental.pallas import tpu as pltpu
from jax.experimental.pallas import tpu_sc as plsc

BATCH = 1024
BW = BATCH // 2
TAU = 30.0
NCLS = 10
NC, NS, L = 2, 16, 16
NW = NC * NS
OUT_PAD = 8192
BPW = OUT_PAD // NW
GRP = 16
NGRP = BPW // GRP
RQ = 4

_GATE_K = (
    (0, 0, 0, 0), (0, 0, 0, 1), (0, 1, 0, -1), (0, 1, 0, 0),
    (0, 0, 1, -1), (0, 0, 1, 0), (0, 1, 1, -2), (0, 1, 1, -1),
    (1, -1, -1, 1), (1, -1, -1, 2), (1, 0, -1, 0), (1, 0, -1, 1),
    (1, -1, 0, 0), (1, -1, 0, 1), (1, 0, 0, -1), (1, 0, 0, 0),
)


def _coef_tc(wall):
    def body(w_ref, k_ref, o_ref):
        w = w_ref[...]
        m = jnp.max(w, axis=-1, keepdims=True)
        e = jnp.exp(w - m)
        p = e / jnp.sum(e, axis=-1, keepdims=True)
        o_ref[...] = jax.lax.dot(p, k_ref[...], precision=lax.Precision.HIGHEST)

    n = wall.shape[0]
    blk = 2048
    return pl.pallas_call(
        body,
        grid=(n // blk,),
        in_specs=[
            pl.BlockSpec((blk, 16), lambda i: (i, 0)),
            pl.BlockSpec((16, 4), lambda i: (0, 0)),
        ],
        out_specs=pl.BlockSpec((blk, 4), lambda i: (i, 0)),
        out_shape=jax.ShapeDtypeStruct((n, 4), jnp.float32),
    )(wall, jnp.asarray(_GATE_K, dtype=jnp.float32))


def _sc_layer(table, aidx, bidx, cfs):
    mesh = plsc.VectorSubcoreMesh(core_axis_name="c", subcore_axis_name="s")

    def body(tab, ai, bi, cf, out, aiv, biv, cfv,
             abufs, bbufs, obufs, sems_a, sems_b, sems_o):
        wid = lax.axis_index("s") * NC + lax.axis_index("c")
        base = wid * BPW
        pltpu.sync_copy(ai.at[pl.ds(base, BPW)], aiv)
        pltpu.sync_copy(bi.at[pl.ds(base, BPW)], biv)
        pltpu.sync_copy(cf.at[pl.ds(base, BPW)], cfv)
        cfvb = cfv.bitcast(jnp.bfloat16)

        def issue(g):
            s = g % 2
            r0 = g * GRP
            cpa = pltpu.async_copy(
                tab.at[aiv.at[pl.ds(r0, GRP)]], abufs[s], sems_a[s])
            cpb = pltpu.async_copy(
                tab.at[biv.at[pl.ds(r0, GRP)]], bbufs[s], sems_b[s])
            return cpa, cpb

        pend = {0: issue(0)}
        out_pend = {}
        for g in range(NGRP):
            s = g % 2
            if g + 1 < NGRP:
                pend[g + 1] = issue(g + 1)
            cpa, cpb = pend.pop(g)
            cpa.wait()
            cpb.wait()
            if g >= 2:
                out_pend.pop(g - 2).wait()
            abufb = abufs[s].bitcast(jnp.bfloat16)
            bbufb = bbufs[s].bitcast(jnp.bfloat16)
            obufb = obufs[s].bitcast(jnp.bfloat16)
            r0 = g * GRP
            for q in range(GRP // RQ):
                rows = [q * RQ + i for i in range(RQ)]
                cs = [tuple(
                    cfvb[pl.ds(2 * (r0 + r), 2), pl.ds(k * L, L)]
                    for k in range(4)) for r in rows]

                def col_fn(j, carry2, rows=rows, cs=cs,
                           abufb=abufb, bbufb=bbufb, obufb=obufb):
                    sl = pl.ds(j * L, L)
                    for r, (c0, c1, c2, c3) in zip(rows, cs):
                        rs = pl.ds(2 * r, 2)
                        av = abufb[rs, sl]
                        bv = bbufb[rs, sl]
                        obufb[rs, sl] = (c0 + c1 * av) + (c2 + c3 * av) * bv
                    return carry2

                lax.fori_loop(0, BW // L, col_fn, 0)
            out_pend[g] = pltpu.async_copy(
                obufs[s], out.at[pl.ds(base + r0, GRP)], sems_o[s])
        for g in sorted(out_pend):
            out_pend.pop(g).wait()

    kfn = pl.kernel(
        body,
        out_type=jax.ShapeDtypeStruct((OUT_PAD, BW), jnp.int32),
        mesh=mesh,
        scratch_types=[
            pltpu.VMEM((BPW,), jnp.int32),
            pltpu.VMEM((BPW,), jnp.int32),
            pltpu.VMEM((BPW, 4 * L), jnp.int32),
            [pltpu.VMEM((GRP, BW), jnp.int32)] * 2,
            [pltpu.VMEM((GRP, BW), jnp.int32)] * 2,
            [pltpu.VMEM((GRP, BW), jnp.int32)] * 2,
            [pltpu.SemaphoreType.DMA] * 2,
            [pltpu.SemaphoreType.DMA] * 2,
            [pltpu.SemaphoreType.DMA] * 2,
        ],
    )
    return kfn(table, aidx, bidx, cfs)


def _gsum_tc(y, n_valid):
    rows = n_valid // NCLS

    def body(y_ref, o_ref):
        yw = y_ref[...]
        lo = jax.lax.bitcast_convert_type(yw << 16, jnp.float32)
        hi = jax.lax.bitcast_convert_type(
            yw & jnp.int32(-65536), jnp.float32)
        o_ref[0, 0, :] = jnp.sum(lo, axis=0) / TAU
        o_ref[0, 1, :] = jnp.sum(hi, axis=0) / TAU

    return pl.pallas_call(
        body,
        grid=(NCLS,),
        in_specs=[pl.BlockSpec((rows, BW), lambda c: (c, 0))],
        out_specs=pl.BlockSpec((1, 2, BW), lambda c: (c, 0, 0)),
        out_shape=jax.ShapeDtypeStruct((NCLS, 2, BW), jnp.float32),
    )(y)


def kernel(x, w1, w2, w3, a1, b1, a2, b2, a3, b3):
    xtb = x.T.astype(jnp.bfloat16)
    xt = jax.lax.bitcast_convert_type(
        xtb.reshape(x.shape[1], BW, 2), jnp.int32)
    n3 = w3.shape[0]
    w3p = jnp.concatenate([w3, jnp.zeros((OUT_PAD - n3, 16), jnp.float32)], 0)
    wall = jnp.concatenate([w1, w2, w3p], axis=0)
    coefs = _coef_tc(wall)
    n = coefs.shape[0]
    cfb = jnp.broadcast_to(
        coefs.astype(jnp.bfloat16)[:, :, None], (n, 4, 2 * L))
    cfp = jax.lax.bitcast_convert_type(
        cfb.reshape(n, 4, L, 2), jnp.int32).reshape(n, 4 * L)
    cf1 = cfp[:OUT_PAD]
    cf2 = cfp[OUT_PAD:2 * OUT_PAD]
    cf3 = cfp[2 * OUT_PAD:]
    padi = jnp.zeros((OUT_PAD - n3,), jnp.int32)
    a3p = jnp.concatenate([a3, padi])
    b3p = jnp.concatenate([b3, padi])
    y1 = _sc_layer(xt, a1, b1, cf1)
    y2 = _sc_layer(y1, a2, b2, cf2)
    y3 = _sc_layer(y2, a3p, b3p, cf3)
    cls = _gsum_tc(y3, n3)
    return cls.transpose(2, 1, 0).reshape(BATCH, NCLS)


# speedup vs baseline: 1.9053x; 1.3996x over previous
import jax
import jax.numpy as jnp
from jax import lax
from jax.experimental import pallas as pl
from jax.experimental.pallas import tpu as pltpu
from jax.experimental.pallas import tpu_sc as plsc

BATCH = 1024
TAU = 30.0
NCLS = 10
NC, NS, L = 2, 16, 16
NW = NC * NS
OUT_PAD = 8192
BPW = OUT_PAD // NW
GRP = 16
NGRP = BPW // GRP
RQ = 8

_GATE_K = (
    (0, 0, 0, 0), (0, 0, 0, 1), (0, 1, 0, -1), (0, 1, 0, 0),
    (0, 0, 1, -1), (0, 0, 1, 0), (0, 1, 1, -2), (0, 1, 1, -1),
    (1, -1, -1, 1), (1, -1, -1, 2), (1, 0, -1, 0), (1, 0, -1, 1),
    (1, -1, 0, 0), (1, -1, 0, 1), (1, 0, 0, -1), (1, 0, 0, 0),
)


def _coef_tc(wall):
    def body(w_ref, k_ref, o_ref):
        w = w_ref[...]
        m = jnp.max(w, axis=-1, keepdims=True)
        e = jnp.exp(w - m)
        p = e / jnp.sum(e, axis=-1, keepdims=True)
        o_ref[...] = jax.lax.dot(p, k_ref[...], precision=lax.Precision.HIGHEST)

    n = wall.shape[0]
    blk = 2048
    return pl.pallas_call(
        body,
        grid=(n // blk,),
        in_specs=[
            pl.BlockSpec((blk, 16), lambda i: (i, 0)),
            pl.BlockSpec((16, 4), lambda i: (0, 0)),
        ],
        out_specs=pl.BlockSpec((blk, 4), lambda i: (i, 0)),
        out_shape=jax.ShapeDtypeStruct((n, 4), jnp.float32),
    )(wall, jnp.asarray(_GATE_K, dtype=jnp.float32))


def _sc_layer(table, aidx, bidx, cfs):
    mesh = plsc.VectorSubcoreMesh(core_axis_name="c", subcore_axis_name="s")

    def body(tab, ai, bi, cf, out, aiv, biv, cfv,
             abufs, bbufs, obufs, sems_a, sems_b, sems_o):
        wid = lax.axis_index("s") * NC + lax.axis_index("c")
        base = wid * BPW
        pltpu.sync_copy(ai.at[pl.ds(base, BPW)], aiv)
        pltpu.sync_copy(bi.at[pl.ds(base, BPW)], biv)
        pltpu.sync_copy(cf.at[:, pl.ds(base, BPW)], cfv)

        def issue(g):
            s = g % 2
            r0 = g * GRP
            cpa = pltpu.async_copy(
                tab.at[aiv.at[pl.ds(r0, GRP)]], abufs[s], sems_a[s])
            cpb = pltpu.async_copy(
                tab.at[biv.at[pl.ds(r0, GRP)]], bbufs[s], sems_b[s])
            return cpa, cpb

        pend = {0: issue(0)}
        out_pend = {}
        for g in range(NGRP):
            s = g % 2
            if g + 1 < NGRP:
                pend[g + 1] = issue(g + 1)
            cpa, cpb = pend.pop(g)
            cpa.wait()
            cpb.wait()
            if g >= 2:
                out_pend.pop(g - 2).wait()
            abuf, bbuf, obuf = abufs[s], bbufs[s], obufs[s]
            r0 = g * GRP
            c0v = cfv[0, pl.ds(r0, GRP)]
            c1v = cfv[1, pl.ds(r0, GRP)]
            c2v = cfv[2, pl.ds(r0, GRP)]
            c3v = cfv[3, pl.ds(r0, GRP)]
            for q in range(GRP // RQ):
                rows = [q * RQ + i for i in range(RQ)]
                cs = [(c0v[r], c1v[r], c2v[r], c3v[r]) for r in rows]

                def col_fn(j, carry2, rows=rows, cs=cs,
                           abuf=abuf, bbuf=bbuf, obuf=obuf):
                    sl = pl.ds(j * L, L)
                    for r, (c0, c1, c2, c3) in zip(rows, cs):
                        av = abuf[r, sl]
                        bv = bbuf[r, sl]
                        obuf[r, sl] = (c0 + c1 * av) + (c2 + c3 * av) * bv
                    return carry2

                lax.fori_loop(0, BATCH // L, col_fn, 0)
            out_pend[g] = pltpu.async_copy(
                obuf, out.at[pl.ds(base + r0, GRP)], sems_o[s])
        for g in sorted(out_pend):
            out_pend.pop(g).wait()

    kfn = pl.kernel(
        body,
        out_type=jax.ShapeDtypeStruct((OUT_PAD, BATCH), jnp.float32),
        mesh=mesh,
        scratch_types=[
            pltpu.VMEM((BPW,), jnp.int32),
            pltpu.VMEM((BPW,), jnp.int32),
            pltpu.VMEM((4, BPW), jnp.float32),
            [pltpu.VMEM((GRP, BATCH), jnp.float32)] * 2,
            [pltpu.VMEM((GRP, BATCH), jnp.float32)] * 2,
            [pltpu.VMEM((GRP, BATCH), jnp.float32)] * 2,
            [pltpu.SemaphoreType.DMA] * 2,
            [pltpu.SemaphoreType.DMA] * 2,
            [pltpu.SemaphoreType.DMA] * 2,
        ],
    )
    return kfn(table, aidx, bidx, cfs)


def _gsum_tc(y, n_valid):
    rows = n_valid // NCLS

    def body(y_ref, o_ref):
        o_ref[...] = (jnp.sum(y_ref[...], axis=0, keepdims=True) / TAU)[None]

    return pl.pallas_call(
        body,
        grid=(NCLS,),
        in_specs=[pl.BlockSpec((rows, BATCH), lambda c: (c, 0))],
        out_specs=pl.BlockSpec((1, 1, BATCH), lambda c: (c, 0, 0)),
        out_shape=jax.ShapeDtypeStruct((NCLS, 1, BATCH), jnp.float32),
    )(y)


def kernel(x, w1, w2, w3, a1, b1, a2, b2, a3, b3):
    xt = x.T
    n3 = w3.shape[0]
    w3p = jnp.concatenate([w3, jnp.zeros((OUT_PAD - n3, 16), jnp.float32)], 0)
    wall = jnp.concatenate([w1, w2, w3p], axis=0)
    coefs = _coef_tc(wall).T
    cf1 = coefs[:, :OUT_PAD]
    cf2 = coefs[:, OUT_PAD:2 * OUT_PAD]
    cf3 = coefs[:, 2 * OUT_PAD:]
    padi = jnp.arange(OUT_PAD - n3, dtype=jnp.int32)
    a3p = jnp.concatenate([a3, padi])
    b3p = jnp.concatenate([b3, padi])
    y1 = _sc_layer(xt, a1, b1, cf1)
    y2 = _sc_layer(y1, a2, b2, cf2)
    y3 = _sc_layer(y2, a3p, b3p, cf3)
    cls = _gsum_tc(y3, n3)
    return cls.reshape(NCLS, BATCH).T


# speedup vs baseline: 2.0273x; 1.0641x over previous
import jax
import jax.numpy as jnp
from jax import lax
from jax.experimental import pallas as pl
from jax.experimental.pallas import tpu as pltpu
from jax.experimental.pallas import tpu_sc as plsc

BATCH = 1024
TAU = 30.0
NCLS = 10
NC, NS, L = 2, 16, 16
NW = NC * NS
OUT_PAD = 8192
BPW = OUT_PAD // NW
GRP = 16
NGRP = BPW // GRP
RQ = 8

_GATE_K = (
    (0, 0, 0, 0), (0, 0, 0, 1), (0, 1, 0, -1), (0, 1, 0, 0),
    (0, 0, 1, -1), (0, 0, 1, 0), (0, 1, 1, -2), (0, 1, 1, -1),
    (1, -1, -1, 1), (1, -1, -1, 2), (1, 0, -1, 0), (1, 0, -1, 1),
    (1, -1, 0, 0), (1, -1, 0, 1), (1, 0, 0, -1), (1, 0, 0, 0),
)


def _coef_tc(wall):
    def body(w_ref, k_ref, o_ref):
        w = w_ref[...]
        m = jnp.max(w, axis=-1, keepdims=True)
        e = jnp.exp(w - m)
        p = e / jnp.sum(e, axis=-1, keepdims=True)
        o_ref[...] = jax.lax.dot(p, k_ref[...], precision=lax.Precision.HIGHEST)

    n = wall.shape[0]
    blk = 2048
    return pl.pallas_call(
        body,
        grid=(n // blk,),
        in_specs=[
            pl.BlockSpec((blk, 16), lambda i: (i, 0)),
            pl.BlockSpec((16, 4), lambda i: (0, 0)),
        ],
        out_specs=pl.BlockSpec((blk, 4), lambda i: (i, 0)),
        out_shape=jax.ShapeDtypeStruct((n, 4), jnp.float32),
    )(wall, jnp.asarray(_GATE_K, dtype=jnp.float32))


def _sc_layer(table, aidx, bidx, cfs):
    mesh = plsc.VectorSubcoreMesh(core_axis_name="c", subcore_axis_name="s")

    def body(tab, ai, bi, cf, out, aiv, biv, cfv,
             abufs, bbufs, obufs, sems_a, sems_b, sems_o):
        wid = lax.axis_index("s") * NC + lax.axis_index("c")
        base = wid * BPW
        pltpu.sync_copy(ai.at[pl.ds(base, BPW)], aiv)
        pltpu.sync_copy(bi.at[pl.ds(base, BPW)], biv)
        pltpu.sync_copy(cf.at[:, pl.ds(base, BPW)], cfv)

        def issue(g):
            s = g % 2
            r0 = g * GRP
            cpa = pltpu.async_copy(
                tab.at[aiv.at[pl.ds(r0, GRP)]], abufs[s], sems_a[s])
            cpb = pltpu.async_copy(
                tab.at[biv.at[pl.ds(r0, GRP)]], bbufs[s], sems_b[s])
            return cpa, cpb

        pend = {0: issue(0)}
        out_pend = {}
        for g in range(NGRP):
            s = g % 2
            if g + 1 < NGRP:
                pend[g + 1] = issue(g + 1)
            cpa, cpb = pend.pop(g)
            cpa.wait()
            cpb.wait()
            if g >= 2:
                out_pend.pop(g - 2).wait()
            abuf, bbuf, obuf = abufs[s], bbufs[s], obufs[s]
            r0 = g * GRP
            c0v = cfv[0, pl.ds(r0, GRP)]
            c1v = cfv[1, pl.ds(r0, GRP)]
            c2v = cfv[2, pl.ds(r0, GRP)]
            c3v = cfv[3, pl.ds(r0, GRP)]
            for q in range(GRP // RQ):
                rows = [q * RQ + i for i in range(RQ)]
                cs = [(c0v[r], c1v[r], c2v[r], c3v[r]) for r in rows]

                def col_fn(j, carry2, rows=rows, cs=cs,
                           abuf=abuf, bbuf=bbuf, obuf=obuf):
                    sl = pl.ds(j * L, L)
                    for r, (c0, c1, c2, c3) in zip(rows, cs):
                        av = abuf[r, sl]
                        bv = bbuf[r, sl]
                        obuf[r, sl] = (c0 + c1 * av) + (c2 + c3 * av) * bv
                    return carry2

                lax.fori_loop(0, BATCH // L, col_fn, 0)
            out_pend[g] = pltpu.async_copy(
                obuf, out.at[pl.ds(base + r0, GRP)], sems_o[s])
        for g in sorted(out_pend):
            out_pend.pop(g).wait()

    kfn = pl.kernel(
        body,
        out_type=jax.ShapeDtypeStruct((OUT_PAD, BATCH), jnp.float32),
        mesh=mesh,
        scratch_types=[
            pltpu.VMEM((BPW,), jnp.int32),
            pltpu.VMEM((BPW,), jnp.int32),
            pltpu.VMEM((4, BPW), jnp.float32),
            [pltpu.VMEM((GRP, BATCH), jnp.float32)] * 2,
            [pltpu.VMEM((GRP, BATCH), jnp.float32)] * 2,
            [pltpu.VMEM((GRP, BATCH), jnp.float32)] * 2,
            [pltpu.SemaphoreType.DMA] * 2,
            [pltpu.SemaphoreType.DMA] * 2,
            [pltpu.SemaphoreType.DMA] * 2,
        ],
    )
    return kfn(table, aidx, bidx, cfs)


def _sc_layer3_gsum(table, aidx, bidx, cfs):
    mesh = plsc.VectorSubcoreMesh(core_axis_name="c", subcore_axis_name="s")

    def body(tab, ai, bi, cf, out, aiv, biv, cfv, acc,
             abufs, bbufs, sems_a, sems_b):
        wid = lax.axis_index("s") * NC + lax.axis_index("c")
        base = wid * BPW
        pltpu.sync_copy(ai.at[pl.ds(base, BPW)], aiv)
        pltpu.sync_copy(bi.at[pl.ds(base, BPW)], biv)
        pltpu.sync_copy(cf.at[:, pl.ds(base, BPW)], cfv)

        def issue(g):
            s = g % 2
            r0 = g * GRP
            cpa = pltpu.async_copy(
                tab.at[aiv.at[pl.ds(r0, GRP)]], abufs[s], sems_a[s])
            cpb = pltpu.async_copy(
                tab.at[biv.at[pl.ds(r0, GRP)]], bbufs[s], sems_b[s])
            return cpa, cpb

        pend = {0: issue(0)}
        for g in range(NGRP):
            s = g % 2
            if g + 1 < NGRP:
                pend[g + 1] = issue(g + 1)
            cpa, cpb = pend.pop(g)
            cpa.wait()
            cpb.wait()
            abuf, bbuf = abufs[s], bbufs[s]
            r0 = g * GRP
            c0v = cfv[0, pl.ds(r0, GRP)]
            c1v = cfv[1, pl.ds(r0, GRP)]
            c2v = cfv[2, pl.ds(r0, GRP)]
            c3v = cfv[3, pl.ds(r0, GRP)]
            m1v = cfv[4, pl.ds(r0, GRP)]
            for q in range(GRP // RQ):
                rows = [q * RQ + i for i in range(RQ)]
                cs = [(c0v[r], c1v[r], c2v[r], c3v[r], m1v[r]) for r in rows]
                init = (g == 0 and q == 0)

                def col_fn(j, carry2, rows=rows, cs=cs, init=init,
                           abuf=abuf, bbuf=bbuf):
                    sl = pl.ds(j * L, L)
                    if init:
                        r0_, (c0, c1, c2, c3, m1) = rows[0], cs[0]
                        av = abuf[r0_, sl]
                        bv = bbuf[r0_, sl]
                        val = (c0 + c1 * av) + (c2 + c3 * av) * bv
                        sacc = val
                        tacc = m1 * val
                        rest = list(zip(rows[1:], cs[1:]))
                    else:
                        sacc = acc[0, sl]
                        tacc = acc[1, sl]
                        rest = list(zip(rows, cs))
                    for r, (c0, c1, c2, c3, m1) in rest:
                        av = abuf[r, sl]
                        bv = bbuf[r, sl]
                        val = (c0 + c1 * av) + (c2 + c3 * av) * bv
                        sacc = sacc + val
                        tacc = tacc + m1 * val
                    acc[0, sl] = sacc
                    acc[1, sl] = tacc
                    return carry2

                lax.fori_loop(0, BATCH // L, col_fn, 0)
        pltpu.sync_copy(acc, out.at[pl.ds(wid * 2, 2)])

    kfn = pl.kernel(
        body,
        out_type=jax.ShapeDtypeStruct((2 * NW, BATCH), jnp.float32),
        mesh=mesh,
        scratch_types=[
            pltpu.VMEM((BPW,), jnp.int32),
            pltpu.VMEM((BPW,), jnp.int32),
            pltpu.VMEM((5, BPW), jnp.float32),
            pltpu.VMEM((2, BATCH), jnp.float32),
            [pltpu.VMEM((GRP, BATCH), jnp.float32)] * 2,
            [pltpu.VMEM((GRP, BATCH), jnp.float32)] * 2,
            [pltpu.SemaphoreType.DMA] * 2,
            [pltpu.SemaphoreType.DMA] * 2,
        ],
    )
    return kfn(table, aidx, bidx, cfs)


def _combine_tc(partials, sel):
    def body(s_ref, p_ref, o_ref):
        o_ref[...] = jax.lax.dot(
            s_ref[...], p_ref[...], precision=lax.Precision.HIGHEST) / TAU

    return pl.pallas_call(
        body,
        out_shape=jax.ShapeDtypeStruct((NCLS, BATCH), jnp.float32),
    )(sel, partials)


def kernel(x, w1, w2, w3, a1, b1, a2, b2, a3, b3):
    xt = x.T
    n3 = w3.shape[0]
    w3p = jnp.concatenate([w3, jnp.zeros((OUT_PAD - n3, 16), jnp.float32)], 0)
    wall = jnp.concatenate([w1, w2, w3p], axis=0)
    coefs = _coef_tc(wall).T
    cf1 = coefs[:, :OUT_PAD]
    cf2 = coefs[:, OUT_PAD:2 * OUT_PAD]
    cf3 = coefs[:, 2 * OUT_PAD:]
    padi = jnp.arange(OUT_PAD - n3, dtype=jnp.int32)
    a3p = jnp.concatenate([a3, padi])
    b3p = jnp.concatenate([b3, padi])

    gsz = n3 // NCLS
    g = jnp.arange(OUT_PAD)
    c0w = (g // BPW) * BPW // gsz
    m1 = ((g // gsz == c0w + 1) & (c0w < NCLS - 1)).astype(jnp.float32)
    cf3m = jnp.concatenate(
        [cf3 * (g < n3).astype(jnp.float32)[None, :], m1[None, :]], axis=0)
    srows = [[0.0] * (2 * NW) for _ in range(NCLS)]
    for w in range(NW):
        c0 = w * BPW // gsz
        srows[c0][2 * w] += 1.0
        srows[c0][2 * w + 1] -= 1.0
        if c0 + 1 < NCLS:
            srows[c0 + 1][2 * w + 1] += 1.0
    sel = jnp.asarray(srows, dtype=jnp.float32)

    y1 = _sc_layer(xt, a1, b1, cf1)
    y2 = _sc_layer(y1, a2, b2, cf2)
    partials = _sc_layer3_gsum(y2, a3p, b3p, cf3m)
    cls = _combine_tc(partials, sel)
    return cls.T
